# baseline (speedup 1.0000x reference)
import functools

import jax
import jax.numpy as jnp
from jax import lax
from jax.experimental import pallas as pl
from jax.experimental.pallas import tpu as pltpu
from jax.experimental.pallas import tpu_sc as plsc

B, C, H, W = 32, 3, 224, 224
K = 16
S = 16
P = 196
NSTRIP = 14
NNODE = 31
NC = 2

_TBL = [
    [0, 0, 0, 0], [0, 0, 0, 1], [0, 1, 0, -1], [0, 1, 0, 0],
    [0, 0, 1, -1], [0, 0, 1, 0], [0, 1, 1, -2], [0, 1, 1, -1],
    [1, -1, -1, 1], [1, -1, -1, 2], [1, 0, -1, 0], [1, 0, -1, 1],
    [1, -1, 0, 0], [1, -1, 0, 1], [1, 0, 0, -1], [1, 0, 0, 0],
]


def _make_m():
    import numpy as np
    t = np.asarray(_TBL, dtype=np.float32)
    m = np.zeros((32, 128), dtype=np.float32)
    for q in range(8):
        half, j = q // 4, q % 4
        for u in range(16):
            m[half * 16 + u, q * 16:(q + 1) * 16] = t[u, j]
    return m


def _prep_body(w0, w1, w2, w3, w4, m_ref, coeff_ref):
    w = jnp.concatenate(
        [w0[...], w1[...], w2[...], w3[...], w4[...]], axis=0)
    e = jnp.exp(w - jnp.max(w, axis=-1, keepdims=True))
    p = e / jnp.sum(e, axis=-1, keepdims=True)
    p4 = p.reshape(NNODE, K // 2, 2, 16)
    pe = p4[:, :, 0, :].reshape(NNODE * K // 2, 16)
    po = p4[:, :, 1, :].reshape(NNODE * K // 2, 16)
    dn = (((1,), (0,)), ((), ()))
    coeff_ref[...] = (
        lax.dot_general(pe, m_ref[pl.ds(0, 16), :], dn,
                        preferred_element_type=jnp.float32)
        + lax.dot_general(po, m_ref[pl.ds(16, 16), :], dn,
                          preferred_element_type=jnp.float32))


_M = _make_m()

_prep = pl.pallas_call(
    _prep_body,
    out_shape=jax.ShapeDtypeStruct((NNODE * K // 2, 128), jnp.float32),
)


NPAIR = NSTRIP // 2


def _sc_body(x, coeffh, rowh, colh, out_hbm, strip0_v, strip1_v,
             coeff_v, row_v, col_v, outb_v, sem0, sem1):
    b = lax.axis_index("s") * NC + lax.axis_index("c")

    def issue(pair, buf, sem):
        handles = []
        for j in range(2):
            st = 2 * pair + j
            for c in range(C):
                handles.append(pltpu.async_copy(
                    x.at[b, c, pl.ds(st * 16, 16), :],
                    buf.at[pl.ds(j * 48 + c * 16, 16), :], sem))
        return handles

    pend = issue(0, strip0_v, sem0)
    pltpu.sync_copy(coeffh, coeff_v)
    pltpu.sync_copy(rowh, row_v)
    pltpu.sync_copy(colh, col_v)

    for pair in range(NPAIR):
        buf = strip0_v if pair % 2 == 0 else strip1_v
        if pair + 1 < NPAIR:
            nxt = issue(pair + 1,
                        strip1_v if pair % 2 == 0 else strip0_v,
                        sem1 if pair % 2 == 0 else sem0)
        else:
            nxt = None
        for h in pend:
            h.wait()
        pend = nxt

        def k_body(k, carry2, _buf=buf, _pair=pair):
            def comb(av, bv, node):
                base = (node * K + k) * 64
                c0 = coeff_v[pl.ds(base, 16)]
                c1 = coeff_v[pl.ds(base + 16, 16)]
                c2 = coeff_v[pl.ds(base + 32, 16)]
                c3 = coeff_v[pl.ds(base + 48, 16)]
                return ((c0 + c1 * av[0]) + bv[0] * (c2 + c3 * av[0]),
                        (c0 + c1 * av[1]) + bv[1] * (c2 + c3 * av[1]))

            node_off = [0, 16, 24, 28, 30]
            stack = []
            for s in range(S):
                ra = row_v[pl.ds((k * S + s) * 16, 16)]
                ca = col_v[pl.ds((k * S + s) * 16, 16)]
                rb = row_v[pl.ds((K * S + k * S + s) * 16, 16)]
                cb = col_v[pl.ds((K * S + k * S + s) * 16, 16)]
                av = (plsc.load_gather(_buf, [ra, ca]),
                      plsc.load_gather(_buf, [ra + 48, ca]))
                bv = (plsc.load_gather(_buf, [rb, cb]),
                      plsc.load_gather(_buf, [rb + 48, cb]))
                cur, lvl, g = comb(av, bv, s), 1, s
                while g % 2 == 1:
                    cur = comb(stack.pop(), cur, node_off[lvl] + g // 2)
                    lvl, g = lvl + 1, g // 2
                stack.append(cur)
            root = stack.pop()
            assert not stack
            mask = lax.iota(jnp.int32, 16) < 14
            plsc.store_compressed(
                outb_v.at[pl.ds(k * P + (2 * _pair) * 14, 16)], root[0], mask=mask)
            plsc.store_compressed(
                outb_v.at[pl.ds(k * P + (2 * _pair + 1) * 14, 16)], root[1],
                mask=mask)
            return carry2

        lax.fori_loop(0, K, k_body, 0, unroll=2)

    pltpu.sync_copy(outb_v.at[pl.ds(0, K * P)],
                    out_hbm.at[pl.ds(b * (K * P), K * P)])


_sc_main = functools.partial(
    pl.kernel,
    mesh=plsc.VectorSubcoreMesh(core_axis_name="c", subcore_axis_name="s"),
    compiler_params=pltpu.CompilerParams(needs_layout_passes=False,
                                         use_tc_tiling_on_sc=False),
    out_type=jax.ShapeDtypeStruct((B * K * P,), jnp.float32),
    scratch_types=[
        pltpu.VMEM((2 * C * 16, W), jnp.float32),
        pltpu.VMEM((2 * C * 16, W), jnp.float32),
        pltpu.VMEM((NNODE * K * 4 * 16,), jnp.float32),
        pltpu.VMEM((2 * K * S * 16,), jnp.int32),
        pltpu.VMEM((2 * K * S * 16,), jnp.int32),
        pltpu.VMEM((K * P + 16,), jnp.float32),
        pltpu.SemaphoreType.DMA,
        pltpu.SemaphoreType.DMA,
    ],
)(_sc_body)


def kernel(x, w0, w1, w2, w3, w4, a_h, a_w, a_c, b_h, b_w, b_c):
    coeff = _prep(w0, w1, w2, w3, w4, jnp.asarray(_M)).reshape(-1)
    rows = jnp.stack([a_c[:, 0] * 16 + a_h[:, 0],
                      b_c[:, 0] * 16 + b_h[:, 0]])
    cols = jnp.stack([a_w[:, 0], b_w[:, 0]])
    lane = jnp.arange(16, dtype=jnp.int32) * 16
    rowt = jnp.broadcast_to(rows.reshape(-1)[:, None],
                            (2 * K * S, 16)).reshape(-1)
    colt = jnp.minimum(cols.reshape(-1)[:, None] + lane[None, :],
                       W - 1).reshape(-1)
    out = _sc_main(x, coeff, rowt, colt)
    return out.reshape(B, K, P, 1)


# speedup vs baseline: 1.4231x; 1.4231x over previous
import functools

import jax
import jax.numpy as jnp
from jax import lax
from jax.experimental import pallas as pl
from jax.experimental.pallas import tpu as pltpu
from jax.experimental.pallas import tpu_sc as plsc

B, C, H, W = 32, 3, 224, 224
K = 16
S = 16
P = 196
NSTRIP = 14
NNODE = 31
NC = 2

_TBL = [
    [0, 0, 0, 0], [0, 0, 0, 1], [0, 1, 0, -1], [0, 1, 0, 0],
    [0, 0, 1, -1], [0, 0, 1, 0], [0, 1, 1, -2], [0, 1, 1, -1],
    [1, -1, -1, 1], [1, -1, -1, 2], [1, 0, -1, 0], [1, 0, -1, 1],
    [1, -1, 0, 0], [1, -1, 0, 1], [1, 0, 0, -1], [1, 0, 0, 0],
]


def _make_m():
    import numpy as np
    t = np.asarray(_TBL, dtype=np.float32)
    m = np.zeros((32, 128), dtype=np.float32)
    for q in range(8):
        half, j = q // 4, q % 4
        for u in range(16):
            m[half * 16 + u, q * 16:(q + 1) * 16] = t[u, j]
    return m


def _prep_body(w0, w1, w2, w3, w4, m_ref, coeff_ref):
    w = jnp.concatenate(
        [w0[...], w1[...], w2[...], w3[...], w4[...]], axis=0)
    e = jnp.exp(w - jnp.max(w, axis=-1, keepdims=True))
    p = e / jnp.sum(e, axis=-1, keepdims=True)
    p4 = p.reshape(NNODE, K // 2, 2, 16)
    pe = p4[:, :, 0, :].reshape(NNODE * K // 2, 16)
    po = p4[:, :, 1, :].reshape(NNODE * K // 2, 16)
    dn = (((1,), (0,)), ((), ()))
    coeff_ref[...] = (
        lax.dot_general(pe, m_ref[pl.ds(0, 16), :], dn,
                        preferred_element_type=jnp.float32)
        + lax.dot_general(po, m_ref[pl.ds(16, 16), :], dn,
                          preferred_element_type=jnp.float32))


_M = _make_m()

_prep = pl.pallas_call(
    _prep_body,
    out_shape=jax.ShapeDtypeStruct((NNODE * K // 2, 128), jnp.float32),
)


NPAIR = NSTRIP // 2


def _sc_body(x, coeffh, rowh, colh, out_hbm, strip0_v, strip1_v,
             coeff_v, row_v, col_v, outb_v, sem0, sem1):
    b = lax.axis_index("s") * NC + lax.axis_index("c")

    def issue(pair, buf, sem):
        handles = []
        for j in range(2):
            st = 2 * pair + j
            for c in range(C):
                handles.append(pltpu.async_copy(
                    x.at[b, c, pl.ds(st * 16, 16), :],
                    buf.at[pl.ds(j * 48 + c * 16, 16), :], sem))
        return handles

    pend = issue(0, strip0_v, sem0)
    pltpu.sync_copy(coeffh, coeff_v)
    pltpu.sync_copy(rowh, row_v)
    pltpu.sync_copy(colh, col_v)

    for pair in range(NPAIR):
        buf = strip0_v if pair % 2 == 0 else strip1_v
        if pair + 1 < NPAIR:
            nxt = issue(pair + 1,
                        strip1_v if pair % 2 == 0 else strip0_v,
                        sem1 if pair % 2 == 0 else sem0)
        else:
            nxt = None
        for h in pend:
            h.wait()
        pend = nxt

        def k_body(k, carry2, _buf=buf, _pair=pair):
            def comb(av, bv, node):
                base = (node * K + k) * 64
                c0 = coeff_v[pl.ds(base, 16)]
                c1 = coeff_v[pl.ds(base + 16, 16)]
                c2 = coeff_v[pl.ds(base + 32, 16)]
                c3 = coeff_v[pl.ds(base + 48, 16)]
                return ((c0 + c1 * av[0]) + bv[0] * (c2 + c3 * av[0]),
                        (c0 + c1 * av[1]) + bv[1] * (c2 + c3 * av[1]))

            node_off = [0, 16, 24, 28, 30]
            stack = []
            for s in range(S):
                ra = row_v[pl.ds((k * S + s) * 16, 16)]
                ca = col_v[pl.ds((k * S + s) * 16, 16)]
                rb = row_v[pl.ds((K * S + k * S + s) * 16, 16)]
                cb = col_v[pl.ds((K * S + k * S + s) * 16, 16)]
                av = (plsc.load_gather(_buf, [ra, ca]),
                      plsc.load_gather(_buf, [ra + 48, ca]))
                bv = (plsc.load_gather(_buf, [rb, cb]),
                      plsc.load_gather(_buf, [rb + 48, cb]))
                cur, lvl, g = comb(av, bv, s), 1, s
                while g % 2 == 1:
                    cur = comb(stack.pop(), cur, node_off[lvl] + g // 2)
                    lvl, g = lvl + 1, g // 2
                stack.append(cur)
            root = stack.pop()
            assert not stack
            mask = lax.iota(jnp.int32, 16) < 14
            plsc.store_compressed(
                outb_v.at[pl.ds(k * P + (2 * _pair) * 14, 16)], root[0], mask=mask)
            plsc.store_compressed(
                outb_v.at[pl.ds(k * P + (2 * _pair + 1) * 14, 16)], root[1],
                mask=mask)
            return carry2

        lax.fori_loop(0, K, k_body, 0)

    pltpu.sync_copy(outb_v.at[pl.ds(0, K * P)],
                    out_hbm.at[pl.ds(b * (K * P), K * P)])


_sc_main = functools.partial(
    pl.kernel,
    mesh=plsc.VectorSubcoreMesh(core_axis_name="c", subcore_axis_name="s"),
    compiler_params=pltpu.CompilerParams(needs_layout_passes=False),
    out_type=jax.ShapeDtypeStruct((B * K * P,), jnp.float32),
    scratch_types=[
        pltpu.VMEM((2 * C * 16, W), jnp.float32),
        pltpu.VMEM((2 * C * 16, W), jnp.float32),
        pltpu.VMEM((NNODE * K * 4 * 16,), jnp.float32),
        pltpu.VMEM((2 * K * S * 16,), jnp.int32),
        pltpu.VMEM((2 * K * S * 16,), jnp.int32),
        pltpu.VMEM((K * P + 16,), jnp.float32),
        pltpu.SemaphoreType.DMA,
        pltpu.SemaphoreType.DMA,
    ],
)(_sc_body)


def kernel(x, w0, w1, w2, w3, w4, a_h, a_w, a_c, b_h, b_w, b_c):
    coeff = _prep(w0, w1, w2, w3, w4, jnp.asarray(_M)).reshape(-1)
    rows = jnp.stack([a_c[:, 0] * 16 + a_h[:, 0],
                      b_c[:, 0] * 16 + b_h[:, 0]])
    cols = jnp.stack([a_w[:, 0], b_w[:, 0]])
    lane = jnp.arange(16, dtype=jnp.int32) * 16
    rowt = jnp.broadcast_to(rows.reshape(-1)[:, None],
                            (2 * K * S, 16)).reshape(-1)
    colt = jnp.minimum(cols.reshape(-1)[:, None] + lane[None, :],
                       W - 1).reshape(-1)
    out = _sc_main(x, coeff, rowt, colt)
    return out.reshape(B, K, P, 1)
